# baseline (speedup 1.0000x reference)
import functools

import jax
import jax.numpy as jnp
from jax import lax
from jax.experimental import pallas as pl
from jax.experimental.pallas import tpu as pltpu
from jax.experimental.pallas import tpu_sc as plsc

N = 10000
E = 320000
IN_DIM = 120
HID = 200
N_STEPS = 8
CONCAT = IN_DIM + HID
G = 25
GN = 400
F = 100
DT = 128
NC = 2
NS = 16
CW = 128
KTOT = E // CW
MAXSPAN = -(-KTOT // NS)
WS = 40
WP = 4
MSP = WS * WP
KPAD = KTOT + MSP
ACC_ROWS = 10240
RPS = ACC_ROWS // NS
TRASH = N
RB = 1000

f32 = jnp.float32


def _sc_agg_body(h0_hbm, h1_hbm, gidx_hbm, sidx_hbm,
                 zeros_hbm, out_hbm,
                 acc_sh, gidx_v, sidx_v, rows_v, sem):
    cid = lax.axis_index("c")
    sid = lax.axis_index("s")
    r0 = sid * RPS
    w = (cid * NS + sid) * WP

    for r, table_hbm in enumerate((h0_hbm, h1_hbm)):
        pltpu.sync_copy(zeros_hbm, rows_v)
        for k in range(RPS // CW):
            pltpu.sync_copy(rows_v, acc_sh.at[pl.ds(r0 + k * CW, CW)])
        plsc.subcore_barrier()

        for p in range(WP):
            pltpu.sync_copy(gidx_hbm.at[w + p], gidx_v)
            pltpu.sync_copy(sidx_hbm.at[w + p], sidx_v)

            def chunk(j, carry):
                pltpu.async_copy(table_hbm.at[gidx_v.at[j]], rows_v,
                                 sem).wait()
                pltpu.sync_copy(rows_v, acc_sh.at[sidx_v.at[j]], add=True)
                return carry
            lax.fori_loop(0, WS, chunk, 0)

        plsc.subcore_barrier()
        pltpu.sync_copy(acc_sh.at[pl.ds(r0, RPS)],
                        out_hbm.at[r, cid, pl.ds(r0, RPS)])


_sc_agg = pl.kernel(
    _sc_agg_body,
    out_type=jax.ShapeDtypeStruct((2, NC, ACC_ROWS, DT), f32),
    mesh=plsc.VectorSubcoreMesh(core_axis_name="c", subcore_axis_name="s"),
    scratch_types=[
        pltpu.VMEM_SHARED((ACC_ROWS, DT), f32),
        pltpu.VMEM((WS, CW), jnp.int32),
        pltpu.VMEM((WS, CW), jnp.int32),
        pltpu.VMEM((CW, DT), f32),
        pltpu.SemaphoreType.DMA,
    ],
)


def _emit_m(hn, w0t, w1t, b0, b1, t0, t1):
    dot = functools.partial(jnp.dot, preferred_element_type=f32)
    m0 = dot(hn, w0t[...]) + b0[...]
    m1 = dot(hn, w1t[...]) + b1[...]
    pad = jnp.zeros((RB, 2 * CW - HID), f32)
    t0[...] = jnp.stack([m0[:, :CW], m1[:, :CW]])
    t1[...] = jnp.stack([jnp.concatenate([m0[:, CW:], pad], axis=1),
                         jnp.concatenate([m1[:, CW:], pad], axis=1)])


def _gru_body(a0, a1, a2, a3, h0, h1, wih, whh, bih, bhh, w0t, w1t, b0, b1,
              o0, o1, t0, t1):
    dot = functools.partial(jnp.dot, preferred_element_type=f32)
    a = jnp.concatenate([(a0[0] + a1[0])[:, :CW],
                         (a2[0] + a3[0])[:, :HID - CW]], axis=1)
    h = jnp.concatenate([h0[:, :F], h1[:, :F]], axis=1)
    gi = dot(a, wih[...]) + bih[...]
    gh = dot(h, whh[...]) + bhh[...]
    r = jax.nn.sigmoid(gi[:, :HID] + gh[:, :HID])
    z = jax.nn.sigmoid(gi[:, HID:2 * HID] + gh[:, HID:2 * HID])
    n = jnp.tanh(gi[:, 2 * HID:] + r * gh[:, 2 * HID:])
    hn = (1.0 - z) * n + z * h
    ones = jnp.ones((RB, 1), f32)
    zpad = jnp.zeros((RB, DT - F - 1), f32)
    o0[...] = jnp.concatenate([hn[:, :F], ones, zpad], axis=1)
    o1[...] = jnp.concatenate([hn[:, F:], ones, zpad], axis=1)
    _emit_m(hn, w0t, w1t, b0, b1, t0, t1)


def _gru_step(agg, h0, h1, wih, whh, bih, bhh, w0t, w1t, b0, b1):
    nb = N // RB
    full = lambda shape: pl.BlockSpec(shape, lambda i: (0,) * len(shape))
    agg_spec = lambda p: pl.BlockSpec((1, RB, DT), lambda i, p=p: (p, i, 0))
    return pl.pallas_call(
        _gru_body,
        grid=(nb,),
        in_specs=[
            agg_spec(0), agg_spec(1), agg_spec(2), agg_spec(3),
            pl.BlockSpec((RB, DT), lambda i: (i, 0)),
            pl.BlockSpec((RB, DT), lambda i: (i, 0)),
            full((HID, 3 * HID)), full((HID, 3 * HID)),
            full((1, 3 * HID)), full((1, 3 * HID)),
            full((HID, HID)), full((HID, HID)),
            full((1, HID)), full((1, HID)),
        ],
        out_specs=[
            pl.BlockSpec((RB, DT), lambda i: (i, 0)),
            pl.BlockSpec((RB, DT), lambda i: (i, 0)),
            pl.BlockSpec((2, RB, CW), lambda i: (0, i, 0)),
            pl.BlockSpec((2, RB, CW), lambda i: (0, i, 0)),
        ],
        out_shape=[
            jax.ShapeDtypeStruct((N, DT), f32),
            jax.ShapeDtypeStruct((N, DT), f32),
            jax.ShapeDtypeStruct((2, N, CW), f32),
            jax.ShapeDtypeStruct((2, N, CW), f32),
        ],
    )(agg, agg, agg, agg, h0, h1, wih, whh, bih, bhh, w0t, w1t, b0, b1)


def _mtrans_body(h0, h1, w0t, w1t, b0, b1, t0, t1):
    hn = jnp.concatenate([h0[:, :F], h1[:, :F]], axis=1)
    _emit_m(hn, w0t, w1t, b0, b1, t0, t1)


def _mtrans(h0, h1, w0t, w1t, b0, b1):
    nb = N // RB
    full = lambda shape: pl.BlockSpec(shape, lambda i: (0,) * len(shape))
    return pl.pallas_call(
        _mtrans_body,
        grid=(nb,),
        in_specs=[
            pl.BlockSpec((RB, DT), lambda i: (i, 0)),
            pl.BlockSpec((RB, DT), lambda i: (i, 0)),
            full((HID, HID)), full((HID, HID)),
            full((1, HID)), full((1, HID)),
        ],
        out_specs=[
            pl.BlockSpec((2, RB, CW), lambda i: (0, i, 0)),
            pl.BlockSpec((2, RB, CW), lambda i: (0, i, 0)),
        ],
        out_shape=[
            jax.ShapeDtypeStruct((2, N, CW), f32),
            jax.ShapeDtypeStruct((2, N, CW), f32),
        ],
    )(h0, h1, w0t, w1t, b0, b1)


def _pool32(y, length):
    no = (length - 3) // 2 + 1
    p = y[:2 * no].reshape(no, 2, y.shape[1]).max(axis=1)
    b = y[2:2 + 2 * no].reshape(no, 2, y.shape[1])[:, 0, :]
    return jnp.maximum(p, b)


def _pool22(y):
    no = y.shape[0] // 2
    return y[:2 * no].reshape(no, 2, y.shape[1]).max(axis=1)


def _tail_body(x, h0, h1, c1w0, c1w1, c1w2, c1b, c2w, c2b,
               cc1w0, cc1w1, cc1w2, cc1b, cc2w, cc2b,
               myw, myb, mzw, mzb, out):
    dot = functools.partial(jnp.dot, preferred_element_type=f32)
    xg = x[...]
    h = jnp.concatenate([h0[:, :F], h1[:, :F]], axis=1)
    c = jnp.concatenate([xg, h], axis=1)
    y1 = (dot(h[0:398], c1w0[...]) + dot(h[1:399], c1w1[...])
          + dot(h[2:400], c1w2[...]) + c1b[...])
    y1 = _pool32(jax.nn.relu(y1), 398)
    y2 = _pool22(jax.nn.relu(dot(y1, c2w[...]) + c2b[...]))
    z1 = (dot(c[0:398], cc1w0[...]) + dot(c[1:399], cc1w1[...])
          + dot(c[2:400], cc1w2[...]) + cc1b[...])
    z1 = _pool32(jax.nn.relu(z1), 398)
    z2 = _pool22(jax.nn.relu(dot(z1, cc2w[...]) + cc2b[...]))
    my = dot(y2, myw[...]) + myb[...]
    mz = dot(z2, mzw[...]) + mzb[...]
    m = jnp.mean(my * mz, axis=0, keepdims=True)
    out[...] = jnp.broadcast_to(m[None], (1, 8, 256))


def _tail(x, h0, h1, c1w0, c1w1, c1w2, c1b, c2w, c2b,
          cc1w0, cc1w1, cc1w2, cc1b, cc2w, cc2b,
          myw, myb, mzw, mzb):
    full = lambda shape: pl.BlockSpec(shape, lambda g: (0,) * len(shape))
    return pl.pallas_call(
        _tail_body,
        grid=(G,),
        in_specs=[
            pl.BlockSpec((GN, IN_DIM), lambda g: (g, 0)),
            pl.BlockSpec((GN, DT), lambda g: (g, 0)),
            pl.BlockSpec((GN, DT), lambda g: (g, 0)),
            full((HID, HID)), full((HID, HID)), full((HID, HID)),
            full((1, HID)),
            full((HID, HID)), full((1, HID)),
            full((CONCAT, CONCAT)), full((CONCAT, CONCAT)),
            full((CONCAT, CONCAT)), full((1, CONCAT)),
            full((CONCAT, CONCAT)), full((1, CONCAT)),
            full((HID, 256)), full((1, 256)),
            full((CONCAT, 256)), full((1, 256)),
        ],
        out_specs=[pl.BlockSpec((1, 8, 256), lambda g: (g, 0, 0))],
        out_shape=[jax.ShapeDtypeStruct((G, 8, 256), f32)],
    )(x, h0, h1, c1w0, c1w1, c1w2, c1b, c2w, c2b,
      cc1w0, cc1w1, cc1w2, cc1b, cc2w, cc2b,
      myw, myb, mzw, mzb)[0][:, 0, :]


def _head_body(avg, l1w, l1b, f1w, f1b, f2w, f2b, cw, cb, p1w, p1b, p2w, p2b,
               w1w, w1b, w2w, w2b, w3w, w3b, logits, pseudo, worst, ft_out):
    dot = functools.partial(jnp.dot, preferred_element_type=f32)
    relu = jax.nn.relu
    a = avg[...]
    h1 = relu(dot(a, l1w[...]) + l1b[...])
    ft = relu(dot(relu(dot(h1, f1w[...]) + f1b[...]), f2w[...]) + f2b[...])
    logits[...] = dot(ft, cw[...]) + cb[...]
    pseudo[...] = dot(relu(dot(ft, p1w[...]) + p1b[...]), p2w[...]) + p2b[...]
    worst[...] = (dot(relu(dot(relu(dot(ft, w1w[...]) + w1b[...]),
                               w2w[...]) + w2b[...]), w3w[...]) + w3b[...])
    ft_out[...] = ft


def _head(avg, *ws):
    shapes = [w.shape for w in ws]
    full = lambda shape: pl.BlockSpec(shape, lambda: (0,) * len(shape))
    return pl.pallas_call(
        _head_body,
        in_specs=[full((G, 256))] + [full(s) for s in shapes],
        out_specs=[full((G, 2)), full((G, 2)), full((G, 2)), full((G, 128))],
        out_shape=[
            jax.ShapeDtypeStruct((G, 2), f32),
            jax.ShapeDtypeStruct((G, 2), f32),
            jax.ShapeDtypeStruct((G, 2), f32),
            jax.ShapeDtypeStruct((G, 128), f32),
        ],
    )(avg, *ws)


def kernel(x, edge_index, edge_types, W_et, b_et, W_ih, W_hh, b_ih, b_hh,
           conv1_w, conv1_b, conv2_w, conv2_b, cconv1_w, cconv1_b,
           cconv2_w, cconv2_b, mlpy_w, mlpy_b, mlpz_w, mlpz_b,
           l1_w, l1_b, f1_w, f1_b, f2_w, f2_b, cls_w, cls_b,
           p1_w, p1_b, p2_w, p2_b, w1_w, w1_b, w2_w, w2_b, w3_w, w3_b):
    src = edge_index[0]
    dst = edge_index[1]
    et = edge_types
    perm = jnp.argsort(et, stable=True)
    srcp = src[perm]
    dstp = dst[perm]
    etp = et[perm]
    sidx0 = jnp.where(etp == 0, dstp, TRASH)
    sidx1 = jnp.where(etp == 1, dstp, TRASH)
    padi = jnp.zeros((KPAD * CW - E,), jnp.int32)
    padt = jnp.full((KPAD * CW - E,), TRASH, jnp.int32)
    gidx = jnp.concatenate([srcp + etp * N, padi]).reshape(KPAD, CW)
    sidx = jnp.stack([jnp.concatenate([sidx0, padt]),
                      jnp.concatenate([sidx1, padt])]).reshape(NC, KPAD, CW)
    m0 = jnp.sum(1 - etp).astype(jnp.int32)
    ce0 = -(-m0 // CW)
    cs1 = m0 // CW
    k0 = ce0
    k1 = KTOT - cs1
    s_arr = jnp.arange(NS, dtype=jnp.int32)
    l0 = -(-k0 // NS)
    l1 = -(-k1 // NS)
    starts = jnp.stack([s_arr * l0, cs1 + s_arr * l1]).astype(jnp.int32)
    counts = jnp.stack([jnp.clip(k0 - s_arr * l0, 0, l0),
                        jnp.clip(k1 - s_arr * l1, 0, l1)]).astype(jnp.int32)
    span = jnp.arange(MSP, dtype=jnp.int32)
    rows = jnp.where(span < counts[:, :, None],
                     jnp.minimum(starts[:, :, None] + span, KPAD - 1),
                     KPAD - 1)
    gidx_exp = jnp.take(gidx, rows.reshape(-1), axis=0).reshape(
        NC * NS * WP, WS, CW)
    sidx_exp = jnp.concatenate(
        [jnp.take(sidx[0], rows[0].reshape(-1), axis=0),
         jnp.take(sidx[1], rows[1].reshape(-1), axis=0)]).reshape(
        NC * NS * WP, WS, CW)
    zeros_z = jnp.zeros((CW, DT), f32)

    ones_col = jnp.ones((N, 1), f32)
    zpad = jnp.zeros((N, DT - F - 1), f32)
    h0 = jnp.concatenate([x[:, :F], ones_col, zpad], axis=1)
    h1 = jnp.concatenate([x[:, F:IN_DIM], jnp.zeros((N, HID - IN_DIM), f32),
                          ones_col, zpad], axis=1)

    wih = W_ih.T
    whh = W_hh.T
    bih = b_ih.reshape(1, 3 * HID)
    bhh = b_hh.reshape(1, 3 * HID)
    w0t = W_et[0].T
    w1t = W_et[1].T
    b0 = b_et[0].reshape(1, HID)
    b1 = b_et[1].reshape(1, HID)

    t0, t1 = _mtrans(h0, h1, w0t, w1t, b0, b1)
    for _ in range(N_STEPS):
        agg = _sc_agg(t0.reshape(2 * N, CW), t1.reshape(2 * N, CW),
                      gidx_exp, sidx_exp, zeros_z)
        agg4 = agg.reshape(2 * NC, ACC_ROWS, DT)
        h0, h1, t0, t1 = _gru_step(agg4, h0, h1, wih, whh, bih, bhh,
                                   w0t, w1t, b0, b1)

    avg = _tail(x, h0, h1,
                conv1_w[:, :, 0].T, conv1_w[:, :, 1].T, conv1_w[:, :, 2].T,
                conv1_b.reshape(1, HID),
                conv2_w[:, :, 0].T, conv2_b.reshape(1, HID),
                cconv1_w[:, :, 0].T, cconv1_w[:, :, 1].T, cconv1_w[:, :, 2].T,
                cconv1_b.reshape(1, CONCAT),
                cconv2_w[:, :, 0].T, cconv2_b.reshape(1, CONCAT),
                mlpy_w.T, mlpy_b.reshape(1, 256),
                mlpz_w.T, mlpz_b.reshape(1, 256))

    logits, pseudo, worst, ft = _head(
        avg,
        l1_w.T, l1_b.reshape(1, 128),
        f1_w.T, f1_b.reshape(1, 64),
        f2_w.T, f2_b.reshape(1, 128),
        cls_w.T, cls_b.reshape(1, 2),
        p1_w.T, p1_b.reshape(1, 256),
        p2_w.T, p2_b.reshape(1, 2),
        w1_w.T, w1_b.reshape(1, 256),
        w2_w.T, w2_b.reshape(1, 256),
        w3_w.T, w3_b.reshape(1, 2))
    return logits, pseudo, worst, ft


# speedup vs baseline: 1.0016x; 1.0016x over previous
import functools

import jax
import jax.numpy as jnp
from jax import lax
from jax.experimental import pallas as pl
from jax.experimental.pallas import tpu as pltpu
from jax.experimental.pallas import tpu_sc as plsc

N = 10000
E = 320000
IN_DIM = 120
HID = 200
N_STEPS = 8
CONCAT = IN_DIM + HID
G = 25
GN = 400
F = 100
DT = 128
NC = 2
NS = 16
CW = 128
KTOT = E // CW
MAXSPAN = -(-KTOT // NS)
WS = 40
WP = 4
MSP = WS * WP
KPAD = KTOT + MSP
ACC_ROWS = 10240
RPS = ACC_ROWS // NS
TRASH = N
RB = 1000

f32 = jnp.float32


def _sc_agg_body(h0_hbm, h1_hbm, gidx_hbm, sidx_hbm,
                 zeros_hbm, out_hbm,
                 acc_sh, gidx_v, sidx_v, rows_v, rows2_v, sem, sem2):
    cid = lax.axis_index("c")
    sid = lax.axis_index("s")
    r0 = sid * RPS
    w = (cid * NS + sid) * WP

    for r, table_hbm in enumerate((h0_hbm, h1_hbm)):
        pltpu.sync_copy(zeros_hbm, rows_v)
        for k in range(RPS // CW):
            pltpu.sync_copy(rows_v, acc_sh.at[pl.ds(r0 + k * CW, CW)])
        plsc.subcore_barrier()

        for p in range(WP):
            pltpu.sync_copy(gidx_hbm.at[w + p], gidx_v)
            pltpu.sync_copy(sidx_hbm.at[w + p], sidx_v)

            def chunk(i, carry):
                j = 2 * i
                cp1 = pltpu.async_copy(table_hbm.at[gidx_v.at[j]], rows_v,
                                       sem)
                cp2 = pltpu.async_copy(table_hbm.at[gidx_v.at[j + 1]],
                                       rows2_v, sem2)
                cp1.wait()
                pltpu.sync_copy(rows_v, acc_sh.at[sidx_v.at[j]], add=True)
                cp2.wait()
                pltpu.sync_copy(rows2_v, acc_sh.at[sidx_v.at[j + 1]],
                                add=True)
                return carry
            lax.fori_loop(0, WS // 2, chunk, 0)

        plsc.subcore_barrier()
        pltpu.sync_copy(acc_sh.at[pl.ds(r0, RPS)],
                        out_hbm.at[r, cid, pl.ds(r0, RPS)])


_sc_agg = pl.kernel(
    _sc_agg_body,
    out_type=jax.ShapeDtypeStruct((2, NC, ACC_ROWS, DT), f32),
    mesh=plsc.VectorSubcoreMesh(core_axis_name="c", subcore_axis_name="s"),
    scratch_types=[
        pltpu.VMEM_SHARED((ACC_ROWS, DT), f32),
        pltpu.VMEM((WS, CW), jnp.int32),
        pltpu.VMEM((WS, CW), jnp.int32),
        pltpu.VMEM((CW, DT), f32),
        pltpu.VMEM((CW, DT), f32),
        pltpu.SemaphoreType.DMA,
        pltpu.SemaphoreType.DMA,
    ],
)


def _emit_m(hn, w0t, w1t, b0, b1, t0, t1):
    dot = functools.partial(jnp.dot, preferred_element_type=f32)
    m0 = dot(hn, w0t[...]) + b0[...]
    m1 = dot(hn, w1t[...]) + b1[...]
    pad = jnp.zeros((RB, 2 * CW - HID), f32)
    t0[...] = jnp.stack([m0[:, :CW], m1[:, :CW]])
    t1[...] = jnp.stack([jnp.concatenate([m0[:, CW:], pad], axis=1),
                         jnp.concatenate([m1[:, CW:], pad], axis=1)])


def _gru_body(a0, a1, a2, a3, h0, h1, wih, whh, bih, bhh, w0t, w1t, b0, b1,
              o0, o1, t0, t1):
    dot = functools.partial(jnp.dot, preferred_element_type=f32)
    a = jnp.concatenate([(a0[0] + a1[0])[:, :CW],
                         (a2[0] + a3[0])[:, :HID - CW]], axis=1)
    h = jnp.concatenate([h0[:, :F], h1[:, :F]], axis=1)
    gi = dot(a, wih[...]) + bih[...]
    gh = dot(h, whh[...]) + bhh[...]
    r = jax.nn.sigmoid(gi[:, :HID] + gh[:, :HID])
    z = jax.nn.sigmoid(gi[:, HID:2 * HID] + gh[:, HID:2 * HID])
    n = jnp.tanh(gi[:, 2 * HID:] + r * gh[:, 2 * HID:])
    hn = (1.0 - z) * n + z * h
    ones = jnp.ones((RB, 1), f32)
    zpad = jnp.zeros((RB, DT - F - 1), f32)
    o0[...] = jnp.concatenate([hn[:, :F], ones, zpad], axis=1)
    o1[...] = jnp.concatenate([hn[:, F:], ones, zpad], axis=1)
    _emit_m(hn, w0t, w1t, b0, b1, t0, t1)


def _gru_step(agg, h0, h1, wih, whh, bih, bhh, w0t, w1t, b0, b1):
    nb = N // RB
    full = lambda shape: pl.BlockSpec(shape, lambda i: (0,) * len(shape))
    agg_spec = lambda p: pl.BlockSpec((1, RB, DT), lambda i, p=p: (p, i, 0))
    return pl.pallas_call(
        _gru_body,
        grid=(nb,),
        in_specs=[
            agg_spec(0), agg_spec(1), agg_spec(2), agg_spec(3),
            pl.BlockSpec((RB, DT), lambda i: (i, 0)),
            pl.BlockSpec((RB, DT), lambda i: (i, 0)),
            full((HID, 3 * HID)), full((HID, 3 * HID)),
            full((1, 3 * HID)), full((1, 3 * HID)),
            full((HID, HID)), full((HID, HID)),
            full((1, HID)), full((1, HID)),
        ],
        out_specs=[
            pl.BlockSpec((RB, DT), lambda i: (i, 0)),
            pl.BlockSpec((RB, DT), lambda i: (i, 0)),
            pl.BlockSpec((2, RB, CW), lambda i: (0, i, 0)),
            pl.BlockSpec((2, RB, CW), lambda i: (0, i, 0)),
        ],
        out_shape=[
            jax.ShapeDtypeStruct((N, DT), f32),
            jax.ShapeDtypeStruct((N, DT), f32),
            jax.ShapeDtypeStruct((2, N, CW), f32),
            jax.ShapeDtypeStruct((2, N, CW), f32),
        ],
    )(agg, agg, agg, agg, h0, h1, wih, whh, bih, bhh, w0t, w1t, b0, b1)


def _mtrans_body(h0, h1, w0t, w1t, b0, b1, t0, t1):
    hn = jnp.concatenate([h0[:, :F], h1[:, :F]], axis=1)
    _emit_m(hn, w0t, w1t, b0, b1, t0, t1)


def _mtrans(h0, h1, w0t, w1t, b0, b1):
    nb = N // RB
    full = lambda shape: pl.BlockSpec(shape, lambda i: (0,) * len(shape))
    return pl.pallas_call(
        _mtrans_body,
        grid=(nb,),
        in_specs=[
            pl.BlockSpec((RB, DT), lambda i: (i, 0)),
            pl.BlockSpec((RB, DT), lambda i: (i, 0)),
            full((HID, HID)), full((HID, HID)),
            full((1, HID)), full((1, HID)),
        ],
        out_specs=[
            pl.BlockSpec((2, RB, CW), lambda i: (0, i, 0)),
            pl.BlockSpec((2, RB, CW), lambda i: (0, i, 0)),
        ],
        out_shape=[
            jax.ShapeDtypeStruct((2, N, CW), f32),
            jax.ShapeDtypeStruct((2, N, CW), f32),
        ],
    )(h0, h1, w0t, w1t, b0, b1)


def _pool32(y, length):
    no = (length - 3) // 2 + 1
    p = y[:2 * no].reshape(no, 2, y.shape[1]).max(axis=1)
    b = y[2:2 + 2 * no].reshape(no, 2, y.shape[1])[:, 0, :]
    return jnp.maximum(p, b)


def _pool22(y):
    no = y.shape[0] // 2
    return y[:2 * no].reshape(no, 2, y.shape[1]).max(axis=1)


def _tail_body(x, h0, h1, c1w0, c1w1, c1w2, c1b, c2w, c2b,
               cc1w0, cc1w1, cc1w2, cc1b, cc2w, cc2b,
               myw, myb, mzw, mzb, out):
    dot = functools.partial(jnp.dot, preferred_element_type=f32)
    xg = x[...]
    h = jnp.concatenate([h0[:, :F], h1[:, :F]], axis=1)
    c = jnp.concatenate([xg, h], axis=1)
    y1 = (dot(h[0:398], c1w0[...]) + dot(h[1:399], c1w1[...])
          + dot(h[2:400], c1w2[...]) + c1b[...])
    y1 = _pool32(jax.nn.relu(y1), 398)
    y2 = _pool22(jax.nn.relu(dot(y1, c2w[...]) + c2b[...]))
    z1 = (dot(c[0:398], cc1w0[...]) + dot(c[1:399], cc1w1[...])
          + dot(c[2:400], cc1w2[...]) + cc1b[...])
    z1 = _pool32(jax.nn.relu(z1), 398)
    z2 = _pool22(jax.nn.relu(dot(z1, cc2w[...]) + cc2b[...]))
    my = dot(y2, myw[...]) + myb[...]
    mz = dot(z2, mzw[...]) + mzb[...]
    m = jnp.mean(my * mz, axis=0, keepdims=True)
    out[...] = jnp.broadcast_to(m[None], (1, 8, 256))


def _tail(x, h0, h1, c1w0, c1w1, c1w2, c1b, c2w, c2b,
          cc1w0, cc1w1, cc1w2, cc1b, cc2w, cc2b,
          myw, myb, mzw, mzb):
    full = lambda shape: pl.BlockSpec(shape, lambda g: (0,) * len(shape))
    return pl.pallas_call(
        _tail_body,
        grid=(G,),
        in_specs=[
            pl.BlockSpec((GN, IN_DIM), lambda g: (g, 0)),
            pl.BlockSpec((GN, DT), lambda g: (g, 0)),
            pl.BlockSpec((GN, DT), lambda g: (g, 0)),
            full((HID, HID)), full((HID, HID)), full((HID, HID)),
            full((1, HID)),
            full((HID, HID)), full((1, HID)),
            full((CONCAT, CONCAT)), full((CONCAT, CONCAT)),
            full((CONCAT, CONCAT)), full((1, CONCAT)),
            full((CONCAT, CONCAT)), full((1, CONCAT)),
            full((HID, 256)), full((1, 256)),
            full((CONCAT, 256)), full((1, 256)),
        ],
        out_specs=[pl.BlockSpec((1, 8, 256), lambda g: (g, 0, 0))],
        out_shape=[jax.ShapeDtypeStruct((G, 8, 256), f32)],
    )(x, h0, h1, c1w0, c1w1, c1w2, c1b, c2w, c2b,
      cc1w0, cc1w1, cc1w2, cc1b, cc2w, cc2b,
      myw, myb, mzw, mzb)[0][:, 0, :]


def _head_body(avg, l1w, l1b, f1w, f1b, f2w, f2b, cw, cb, p1w, p1b, p2w, p2b,
               w1w, w1b, w2w, w2b, w3w, w3b, logits, pseudo, worst, ft_out):
    dot = functools.partial(jnp.dot, preferred_element_type=f32)
    relu = jax.nn.relu
    a = avg[...]
    h1 = relu(dot(a, l1w[...]) + l1b[...])
    ft = relu(dot(relu(dot(h1, f1w[...]) + f1b[...]), f2w[...]) + f2b[...])
    logits[...] = dot(ft, cw[...]) + cb[...]
    pseudo[...] = dot(relu(dot(ft, p1w[...]) + p1b[...]), p2w[...]) + p2b[...]
    worst[...] = (dot(relu(dot(relu(dot(ft, w1w[...]) + w1b[...]),
                               w2w[...]) + w2b[...]), w3w[...]) + w3b[...])
    ft_out[...] = ft


def _head(avg, *ws):
    shapes = [w.shape for w in ws]
    full = lambda shape: pl.BlockSpec(shape, lambda: (0,) * len(shape))
    return pl.pallas_call(
        _head_body,
        in_specs=[full((G, 256))] + [full(s) for s in shapes],
        out_specs=[full((G, 2)), full((G, 2)), full((G, 2)), full((G, 128))],
        out_shape=[
            jax.ShapeDtypeStruct((G, 2), f32),
            jax.ShapeDtypeStruct((G, 2), f32),
            jax.ShapeDtypeStruct((G, 2), f32),
            jax.ShapeDtypeStruct((G, 128), f32),
        ],
    )(avg, *ws)


def kernel(x, edge_index, edge_types, W_et, b_et, W_ih, W_hh, b_ih, b_hh,
           conv1_w, conv1_b, conv2_w, conv2_b, cconv1_w, cconv1_b,
           cconv2_w, cconv2_b, mlpy_w, mlpy_b, mlpz_w, mlpz_b,
           l1_w, l1_b, f1_w, f1_b, f2_w, f2_b, cls_w, cls_b,
           p1_w, p1_b, p2_w, p2_b, w1_w, w1_b, w2_w, w2_b, w3_w, w3_b):
    src = edge_index[0]
    dst = edge_index[1]
    et = edge_types
    perm = jnp.argsort(et, stable=True)
    srcp = src[perm]
    dstp = dst[perm]
    etp = et[perm]
    sidx0 = jnp.where(etp == 0, dstp, TRASH)
    sidx1 = jnp.where(etp == 1, dstp, TRASH)
    padi = jnp.zeros((KPAD * CW - E,), jnp.int32)
    padt = jnp.full((KPAD * CW - E,), TRASH, jnp.int32)
    gidx = jnp.concatenate([srcp + etp * N, padi]).reshape(KPAD, CW)
    sidx = jnp.stack([jnp.concatenate([sidx0, padt]),
                      jnp.concatenate([sidx1, padt])]).reshape(NC, KPAD, CW)
    m0 = jnp.sum(1 - etp).astype(jnp.int32)
    ce0 = -(-m0 // CW)
    cs1 = m0 // CW
    k0 = ce0
    k1 = KTOT - cs1
    s_arr = jnp.arange(NS, dtype=jnp.int32)
    l0 = -(-k0 // NS)
    l1 = -(-k1 // NS)
    starts = jnp.stack([s_arr * l0, cs1 + s_arr * l1]).astype(jnp.int32)
    counts = jnp.stack([jnp.clip(k0 - s_arr * l0, 0, l0),
                        jnp.clip(k1 - s_arr * l1, 0, l1)]).astype(jnp.int32)
    span = jnp.arange(MSP, dtype=jnp.int32)
    rows = jnp.where(span < counts[:, :, None],
                     jnp.minimum(starts[:, :, None] + span, KPAD - 1),
                     KPAD - 1)
    gidx_exp = jnp.take(gidx, rows.reshape(-1), axis=0).reshape(
        NC * NS * WP, WS, CW)
    sidx_exp = jnp.concatenate(
        [jnp.take(sidx[0], rows[0].reshape(-1), axis=0),
         jnp.take(sidx[1], rows[1].reshape(-1), axis=0)]).reshape(
        NC * NS * WP, WS, CW)
    zeros_z = jnp.zeros((CW, DT), f32)

    ones_col = jnp.ones((N, 1), f32)
    zpad = jnp.zeros((N, DT - F - 1), f32)
    h0 = jnp.concatenate([x[:, :F], ones_col, zpad], axis=1)
    h1 = jnp.concatenate([x[:, F:IN_DIM], jnp.zeros((N, HID - IN_DIM), f32),
                          ones_col, zpad], axis=1)

    wih = W_ih.T
    whh = W_hh.T
    bih = b_ih.reshape(1, 3 * HID)
    bhh = b_hh.reshape(1, 3 * HID)
    w0t = W_et[0].T
    w1t = W_et[1].T
    b0 = b_et[0].reshape(1, HID)
    b1 = b_et[1].reshape(1, HID)

    t0, t1 = _mtrans(h0, h1, w0t, w1t, b0, b1)
    for _ in range(N_STEPS):
        agg = _sc_agg(t0.reshape(2 * N, CW), t1.reshape(2 * N, CW),
                      gidx_exp, sidx_exp, zeros_z)
        agg4 = agg.reshape(2 * NC, ACC_ROWS, DT)
        h0, h1, t0, t1 = _gru_step(agg4, h0, h1, wih, whh, bih, bhh,
                                   w0t, w1t, b0, b1)

    avg = _tail(x, h0, h1,
                conv1_w[:, :, 0].T, conv1_w[:, :, 1].T, conv1_w[:, :, 2].T,
                conv1_b.reshape(1, HID),
                conv2_w[:, :, 0].T, conv2_b.reshape(1, HID),
                cconv1_w[:, :, 0].T, cconv1_w[:, :, 1].T, cconv1_w[:, :, 2].T,
                cconv1_b.reshape(1, CONCAT),
                cconv2_w[:, :, 0].T, cconv2_b.reshape(1, CONCAT),
                mlpy_w.T, mlpy_b.reshape(1, 256),
                mlpz_w.T, mlpz_b.reshape(1, 256))

    logits, pseudo, worst, ft = _head(
        avg,
        l1_w.T, l1_b.reshape(1, 128),
        f1_w.T, f1_b.reshape(1, 64),
        f2_w.T, f2_b.reshape(1, 128),
        cls_w.T, cls_b.reshape(1, 2),
        p1_w.T, p1_b.reshape(1, 256),
        p2_w.T, p2_b.reshape(1, 2),
        w1_w.T, w1_b.reshape(1, 256),
        w2_w.T, w2_b.reshape(1, 256),
        w3_w.T, w3_b.reshape(1, 2))
    return logits, pseudo, worst, ft
